# baseline (device time: 59710 ns/iter reference)
import jax
import jax.numpy as jnp
from jax import lax
from jax.experimental import pallas as pl
from jax.experimental.pallas import tpu as pltpu

N_DEV = 8


def _gelu(y):
    c = 0.7978845608028654
    return 0.5 * y * (1.0 + jnp.tanh(c * (y + 0.044715 * y * y * y)))


def kernel(x, w_mat):
    m, _ = x.shape
    _, n = w_mat.shape
    m_per = m // N_DEV

    def body(x_ref, w_ref, out_ref, acc_ref, comm_ref, send_sems, recv_sems):
        my = lax.axis_index("i")
        left = lax.rem(my + (N_DEV - 1), N_DEV)
        right = lax.rem(my + 1, N_DEV)

        barrier_sem = pltpu.get_barrier_semaphore()
        for nbr in (left, right):
            pl.semaphore_signal(
                barrier_sem, inc=1,
                device_id=(nbr,), device_id_type=pl.DeviceIdType.MESH,
            )
        pl.semaphore_wait(barrier_sem, 2)

        acc_ref[...] = jnp.dot(
            x_ref[...], w_ref[...], preferred_element_type=jnp.float32
        )

        first = lax.rem(my + (N_DEV - 1), N_DEV)
        comm_ref[0, :, :] = acc_ref[pl.ds(first * m_per, m_per), :]

        for s in range(N_DEV - 1):
            rdma = pltpu.make_async_remote_copy(
                src_ref=comm_ref.at[s],
                dst_ref=comm_ref.at[s + 1],
                send_sem=send_sems.at[s],
                recv_sem=recv_sems.at[s],
                device_id=(right,),
                device_id_type=pl.DeviceIdType.MESH,
            )
            rdma.start()
            rdma.wait()

            recv_chunk = lax.rem(my + (2 * N_DEV - s - 2), N_DEV)
            partial = acc_ref[pl.ds(recv_chunk * m_per, m_per), :]
            if s < N_DEV - 2:
                comm_ref[s + 1, :, :] = comm_ref[s + 1, :, :] + partial
            else:
                out_ref[...] = _gelu(comm_ref[s + 1, :, :] + partial)

    return pl.pallas_call(
        body,
        out_shape=jax.ShapeDtypeStruct((m_per, n), jnp.float32),
        in_specs=[
            pl.BlockSpec(memory_space=pltpu.VMEM),
            pl.BlockSpec(memory_space=pltpu.VMEM),
        ],
        out_specs=pl.BlockSpec(memory_space=pltpu.VMEM),
        scratch_shapes=[
            pltpu.VMEM((m, n), jnp.float32),
            pltpu.VMEM((N_DEV, m_per, n), jnp.float32),
            pltpu.SemaphoreType.DMA((N_DEV - 1,)),
            pltpu.SemaphoreType.DMA((N_DEV - 1,)),
        ],
        compiler_params=pltpu.CompilerParams(collective_id=0),
    )(x, w_mat)


# device time: 31171 ns/iter; 1.9156x vs baseline; 1.9156x over previous
import jax
import jax.numpy as jnp
from jax import lax
from jax.experimental import pallas as pl
from jax.experimental.pallas import tpu as pltpu

N_DEV = 8
N_HOP = N_DEV - 1
Q = 2
USE_GRAY = True


def _gelu(y):
    c = 0.7978845608028654
    return 0.5 * y * (1.0 + jnp.tanh(c * (y + 0.044715 * y * y * y)))


def _ring_to_pos(r):
    if USE_GRAY:
        return jnp.where(r < 4, r, 11 - r)
    return r


def kernel(x, w_mat):
    m, _ = x.shape
    _, n = w_mat.shape
    m_per = m // N_DEV
    half = n // 2
    subw = half // Q

    def body(x_ref, w_ref, out_ref, acc_ref, comm_a, comm_b,
             send_a, recv_a, send_b, recv_b):
        my = lax.axis_index("i")
        my_r = _ring_to_pos(my)
        right = _ring_to_pos(lax.rem(my_r + 1, N_DEV))
        left = _ring_to_pos(lax.rem(my_r + (N_DEV - 1), N_DEV))

        def mk(comm, ssem, rsem, s, q, dev):
            cols = pl.ds(q * subw, subw)
            return pltpu.make_async_remote_copy(
                src_ref=comm.at[s, :, cols],
                dst_ref=comm.at[s + 1, :, cols],
                send_sem=ssem.at[s * Q + q],
                recv_sem=rsem.at[s * Q + q],
                device_id=(dev,),
                device_id_type=pl.DeviceIdType.MESH,
            )

        barrier_sem = pltpu.get_barrier_semaphore()
        for nbr in (left, right):
            pl.semaphore_signal(
                barrier_sem, inc=1,
                device_id=(nbr,), device_id_type=pl.DeviceIdType.MESH,
            )
        pl.semaphore_wait(barrier_sem, 2)

        acc_ref[...] = jnp.dot(
            x_ref[...], w_ref[...], preferred_element_type=jnp.float32
        )

        chunk_a0 = _ring_to_pos(lax.rem(my_r + (N_DEV - 1), N_DEV))
        chunk_b0 = _ring_to_pos(lax.rem(my_r + 1, N_DEV))
        comm_a[0, :, :] = acc_ref[pl.ds(chunk_a0 * m_per, m_per), :half]
        comm_b[0, :, :] = acc_ref[pl.ds(chunk_b0 * m_per, m_per), half:]

        for q in range(Q):
            mk(comm_a, send_a, recv_a, 0, q, right).start()
            mk(comm_b, send_b, recv_b, 0, q, left).start()

        for s in range(N_HOP):
            row_a = _ring_to_pos(lax.rem(my_r + (2 * N_DEV - s - 2), N_DEV))
            row_b = _ring_to_pos(lax.rem(my_r + s + 2, N_DEV))
            last = s == N_HOP - 1
            for q in range(Q):
                ca = pl.ds(q * subw, subw)
                cb = pl.ds(half + q * subw, subw)

                mk(comm_a, send_a, recv_a, s, q, right).wait_recv()
                data_a = (comm_a[s + 1, :, ca]
                          + acc_ref[pl.ds(row_a * m_per, m_per), ca])
                if last:
                    out_ref[:, ca] = _gelu(data_a)
                else:
                    comm_a[s + 1, :, ca] = data_a
                    mk(comm_a, send_a, recv_a, s + 1, q, right).start()

                mk(comm_b, send_b, recv_b, s, q, left).wait_recv()
                data_b = (comm_b[s + 1, :, ca]
                          + acc_ref[pl.ds(row_b * m_per, m_per), cb])
                if last:
                    out_ref[:, cb] = _gelu(data_b)
                else:
                    comm_b[s + 1, :, ca] = data_b
                    mk(comm_b, send_b, recv_b, s + 1, q, left).start()

        for s in range(N_HOP):
            for q in range(Q):
                mk(comm_a, send_a, recv_a, s, q, right).wait_send()
                mk(comm_b, send_b, recv_b, s, q, left).wait_send()

    n_sem = N_HOP * Q
    return pl.pallas_call(
        body,
        out_shape=jax.ShapeDtypeStruct((m_per, n), jnp.float32),
        in_specs=[
            pl.BlockSpec(memory_space=pltpu.VMEM),
            pl.BlockSpec(memory_space=pltpu.VMEM),
        ],
        out_specs=pl.BlockSpec(memory_space=pltpu.VMEM),
        scratch_shapes=[
            pltpu.VMEM((m, n), jnp.float32),
            pltpu.VMEM((N_DEV, m_per, half), jnp.float32),
            pltpu.VMEM((N_DEV, m_per, half), jnp.float32),
            pltpu.SemaphoreType.DMA((n_sem,)),
            pltpu.SemaphoreType.DMA((n_sem,)),
            pltpu.SemaphoreType.DMA((n_sem,)),
            pltpu.SemaphoreType.DMA((n_sem,)),
        ],
        compiler_params=pltpu.CompilerParams(collective_id=0),
    )(x, w_mat)


# device time: 29827 ns/iter; 2.0019x vs baseline; 1.0451x over previous
import jax
import jax.numpy as jnp
from jax import lax
from jax.experimental import pallas as pl
from jax.experimental.pallas import tpu as pltpu

N_DEV = 8
N_HOP = N_DEV - 1
Q = 4
USE_GRAY = True


def _gelu(y):
    c = 0.7978845608028654
    return 0.5 * y * (1.0 + jnp.tanh(c * (y + 0.044715 * y * y * y)))


def _ring_to_pos(r):
    if USE_GRAY:
        return jnp.where(r < 4, r, 11 - r)
    return r


def kernel(x, w_mat):
    m, _ = x.shape
    _, n = w_mat.shape
    m_per = m // N_DEV
    half = n // 2
    subw = half // Q

    def body(x_ref, w_ref, out_ref, acc_ref, comm_a, comm_b,
             send_a, recv_a, send_b, recv_b):
        my = lax.axis_index("i")
        my_r = _ring_to_pos(my)
        right = _ring_to_pos(lax.rem(my_r + 1, N_DEV))
        left = _ring_to_pos(lax.rem(my_r + (N_DEV - 1), N_DEV))

        def mk(comm, ssem, rsem, s, q, dev):
            cols = pl.ds(q * subw, subw)
            return pltpu.make_async_remote_copy(
                src_ref=comm.at[s, :, cols],
                dst_ref=comm.at[s + 1, :, cols],
                send_sem=ssem.at[s * Q + q],
                recv_sem=rsem.at[s * Q + q],
                device_id=(dev,),
                device_id_type=pl.DeviceIdType.MESH,
            )

        barrier_sem = pltpu.get_barrier_semaphore()
        for nbr in (left, right):
            pl.semaphore_signal(
                barrier_sem, inc=1,
                device_id=(nbr,), device_id_type=pl.DeviceIdType.MESH,
            )
        pl.semaphore_wait(barrier_sem, 2)

        acc_ref[...] = jnp.dot(
            x_ref[...], w_ref[...], preferred_element_type=jnp.float32
        )

        chunk_a0 = _ring_to_pos(lax.rem(my_r + (N_DEV - 1), N_DEV))
        chunk_b0 = _ring_to_pos(lax.rem(my_r + 1, N_DEV))
        comm_a[0, :, :] = acc_ref[pl.ds(chunk_a0 * m_per, m_per), :half]
        comm_b[0, :, :] = acc_ref[pl.ds(chunk_b0 * m_per, m_per), half:]

        for q in range(Q):
            mk(comm_a, send_a, recv_a, 0, q, right).start()
            mk(comm_b, send_b, recv_b, 0, q, left).start()

        for s in range(N_HOP):
            row_a = _ring_to_pos(lax.rem(my_r + (2 * N_DEV - s - 2), N_DEV))
            row_b = _ring_to_pos(lax.rem(my_r + s + 2, N_DEV))
            last = s == N_HOP - 1
            for q in range(Q):
                ca = pl.ds(q * subw, subw)
                cb = pl.ds(half + q * subw, subw)

                mk(comm_a, send_a, recv_a, s, q, right).wait_recv()
                data_a = (comm_a[s + 1, :, ca]
                          + acc_ref[pl.ds(row_a * m_per, m_per), ca])
                if last:
                    out_ref[:, ca] = _gelu(data_a)
                else:
                    comm_a[s + 1, :, ca] = data_a
                    mk(comm_a, send_a, recv_a, s + 1, q, right).start()

                mk(comm_b, send_b, recv_b, s, q, left).wait_recv()
                data_b = (comm_b[s + 1, :, ca]
                          + acc_ref[pl.ds(row_b * m_per, m_per), cb])
                if last:
                    out_ref[:, cb] = _gelu(data_b)
                else:
                    comm_b[s + 1, :, ca] = data_b
                    mk(comm_b, send_b, recv_b, s + 1, q, left).start()

        for s in range(N_HOP):
            for q in range(Q):
                mk(comm_a, send_a, recv_a, s, q, right).wait_send()
                mk(comm_b, send_b, recv_b, s, q, left).wait_send()

    n_sem = N_HOP * Q
    return pl.pallas_call(
        body,
        out_shape=jax.ShapeDtypeStruct((m_per, n), jnp.float32),
        in_specs=[
            pl.BlockSpec(memory_space=pltpu.VMEM),
            pl.BlockSpec(memory_space=pltpu.VMEM),
        ],
        out_specs=pl.BlockSpec(memory_space=pltpu.VMEM),
        scratch_shapes=[
            pltpu.VMEM((m, n), jnp.float32),
            pltpu.VMEM((N_DEV, m_per, half), jnp.float32),
            pltpu.VMEM((N_DEV, m_per, half), jnp.float32),
            pltpu.SemaphoreType.DMA((n_sem,)),
            pltpu.SemaphoreType.DMA((n_sem,)),
            pltpu.SemaphoreType.DMA((n_sem,)),
            pltpu.SemaphoreType.DMA((n_sem,)),
        ],
        compiler_params=pltpu.CompilerParams(collective_id=0),
    )(x, w_mat)


# device time: 29253 ns/iter; 2.0412x vs baseline; 1.0196x over previous
import jax
import jax.numpy as jnp
from jax import lax
from jax.experimental import pallas as pl
from jax.experimental.pallas import tpu as pltpu

N_DEV = 8
N_HOP = N_DEV - 1
Q = 4
USE_GRAY = True


def _gelu(y):
    c = 0.7978845608028654
    return 0.5 * y * (1.0 + jnp.tanh(c * (y + 0.044715 * y * y * y)))


def _ring_to_pos(r):
    if USE_GRAY:
        return jnp.where(r < 4, r, 11 - r)
    return r


def kernel(x, w_mat):
    m, _ = x.shape
    _, n = w_mat.shape
    m_per = m // N_DEV
    half = n // 2
    m_sub = m_per // Q

    def body(x_ref, w_ref, out_ref, acc_ref, comm_a, comm_b,
             send_a, recv_a, send_b, recv_b):
        my = lax.axis_index("i")
        my_r = _ring_to_pos(my)
        right = _ring_to_pos(lax.rem(my_r + 1, N_DEV))
        left = _ring_to_pos(lax.rem(my_r + (N_DEV - 1), N_DEV))

        def mk(comm, ssem, rsem, s, q, dev):
            rows = pl.ds(q * m_sub, m_sub)
            return pltpu.make_async_remote_copy(
                src_ref=comm.at[s, rows, :],
                dst_ref=comm.at[s + 1, rows, :],
                send_sem=ssem.at[s * Q + q],
                recv_sem=rsem.at[s * Q + q],
                device_id=(dev,),
                device_id_type=pl.DeviceIdType.MESH,
            )

        barrier_sem = pltpu.get_barrier_semaphore()
        for nbr in (left, right):
            pl.semaphore_signal(
                barrier_sem, inc=1,
                device_id=(nbr,), device_id_type=pl.DeviceIdType.MESH,
            )
        pl.semaphore_wait(barrier_sem, 2)

        chunk_a0 = _ring_to_pos(lax.rem(my_r + (N_DEV - 1), N_DEV))
        chunk_b0 = _ring_to_pos(lax.rem(my_r + 1, N_DEV))

        comm_a[0, :, :] = jnp.dot(
            x_ref[pl.ds(chunk_a0 * m_per, m_per), :], w_ref[:, :half],
            preferred_element_type=jnp.float32,
        )
        comm_b[0, :, :] = jnp.dot(
            x_ref[pl.ds(chunk_b0 * m_per, m_per), :], w_ref[:, half:],
            preferred_element_type=jnp.float32,
        )
        for q in range(Q):
            mk(comm_a, send_a, recv_a, 0, q, right).start()
            mk(comm_b, send_b, recv_b, 0, q, left).start()

        acc_ref[...] = jnp.dot(
            x_ref[...], w_ref[...], preferred_element_type=jnp.float32
        )

        for s in range(N_HOP):
            row_a = _ring_to_pos(lax.rem(my_r + (2 * N_DEV - s - 2), N_DEV))
            row_b = _ring_to_pos(lax.rem(my_r + s + 2, N_DEV))
            last = s == N_HOP - 1
            for q in range(Q):
                rows = pl.ds(q * m_sub, m_sub)

                mk(comm_a, send_a, recv_a, s, q, right).wait_recv()
                data_a = (comm_a[s + 1, rows, :]
                          + acc_ref[pl.ds(row_a * m_per + q * m_sub, m_sub),
                                    :half])
                if last:
                    out_ref[rows, :half] = _gelu(data_a)
                else:
                    comm_a[s + 1, rows, :] = data_a
                    mk(comm_a, send_a, recv_a, s + 1, q, right).start()

                mk(comm_b, send_b, recv_b, s, q, left).wait_recv()
                data_b = (comm_b[s + 1, rows, :]
                          + acc_ref[pl.ds(row_b * m_per + q * m_sub, m_sub),
                                    half:])
                if last:
                    out_ref[rows, half:] = _gelu(data_b)
                else:
                    comm_b[s + 1, rows, :] = data_b
                    mk(comm_b, send_b, recv_b, s + 1, q, left).start()

        for s in range(N_HOP):
            for q in range(Q):
                mk(comm_a, send_a, recv_a, s, q, right).wait_send()
                mk(comm_b, send_b, recv_b, s, q, left).wait_send()

    n_sem = N_HOP * Q
    return pl.pallas_call(
        body,
        out_shape=jax.ShapeDtypeStruct((m_per, n), jnp.float32),
        in_specs=[
            pl.BlockSpec(memory_space=pltpu.VMEM),
            pl.BlockSpec(memory_space=pltpu.VMEM),
        ],
        out_specs=pl.BlockSpec(memory_space=pltpu.VMEM),
        scratch_shapes=[
            pltpu.VMEM((m, n), jnp.float32),
            pltpu.VMEM((N_DEV, m_per, half), jnp.float32),
            pltpu.VMEM((N_DEV, m_per, half), jnp.float32),
            pltpu.SemaphoreType.DMA((n_sem,)),
            pltpu.SemaphoreType.DMA((n_sem,)),
            pltpu.SemaphoreType.DMA((n_sem,)),
            pltpu.SemaphoreType.DMA((n_sem,)),
        ],
        compiler_params=pltpu.CompilerParams(collective_id=0),
    )(x, w_mat)


# device time: 24529 ns/iter; 2.4343x vs baseline; 1.1926x over previous
import jax
import jax.numpy as jnp
from jax import lax
from jax.experimental import pallas as pl
from jax.experimental.pallas import tpu as pltpu

N_DEV = 8
M_PER = 128

GROUP_COLS = [(0, 384), (384, 384), (768, 256)]
GROUP_MASKS = [(4, 2, 1), (2, 1, 4), (1, 4, 2)]

STEP_BASE = (0, 4, 6)
N_SLOT = 7


def _steps(masks):
    m1, m2, m3 = masks
    return [(m1, (m2, m2 ^ m3, m3, 0)), (m2, (m3, 0)), (m3, (0,))]


def _gelu(y):
    c = 0.7978845608028654
    return 0.5 * y * (1.0 + jnp.tanh(c * (y + 0.044715 * y * y * y)))


def _u_of_p(p):
    return p ^ ((p >> 1) & 1)


def kernel(x, w_mat):
    m, _ = x.shape
    _, n = w_mat.shape
    m_per = m // N_DEV

    def body(x_ref, w_ref, out_ref,
             acc_a, acc_b, acc_c, buf_a, buf_b, buf_c,
             ss_a, rs_a, ss_b, rs_b, ss_c, rs_c):
        accs = (acc_a, acc_b, acc_c)
        bufs = (buf_a, buf_b, buf_c)
        ssems = (ss_a, ss_b, ss_c)
        rsems = (rs_a, rs_b, rs_c)

        my = lax.axis_index("i")
        u = _u_of_p(my)

        def rows_of_chunk(cu):
            return pl.ds(_u_of_p(cu) * m_per, m_per)

        def mk(g, step, j, s):
            md, slots = _steps(GROUP_MASKS[g])[step]
            idx = STEP_BASE[step] + j
            partner = _u_of_p(u ^ md)
            return pltpu.make_async_remote_copy(
                src_ref=accs[g].at[rows_of_chunk(u ^ md ^ s), :],
                dst_ref=bufs[g].at[idx],
                send_sem=ssems[g].at[idx],
                recv_sem=rsems[g].at[idx],
                device_id=(partner,),
                device_id_type=pl.DeviceIdType.MESH,
            )

        def recv_add(g, step, j, s):
            idx = STEP_BASE[step] + j
            mk(g, step, j, s).wait_recv()
            r = rows_of_chunk(u ^ s)
            accs[g][r, :] = accs[g][r, :] + bufs[g][idx, :, :]

        barrier_sem = pltpu.get_barrier_semaphore()
        for md in (1, 2, 4):
            pl.semaphore_signal(
                barrier_sem, inc=1,
                device_id=(_u_of_p(u ^ md),),
                device_id_type=pl.DeviceIdType.MESH,
            )
        pl.semaphore_wait(barrier_sem, 3)

        for g, (c0, w) in enumerate(GROUP_COLS):
            accs[g][...] = jnp.dot(
                x_ref[...], w_ref[:, c0:c0 + w],
                preferred_element_type=jnp.float32,
            )
            md, slots = _steps(GROUP_MASKS[g])[0]
            for j, s in enumerate(slots):
                mk(g, 0, j, s).start()

        for g in range(3):
            _, slots1 = _steps(GROUP_MASKS[g])[0]
            recv_add(g, 0, 0, slots1[0])
            recv_add(g, 0, 1, slots1[1])
            _, slots2 = _steps(GROUP_MASKS[g])[1]
            for j, s in enumerate(slots2):
                mk(g, 1, j, s).start()

        for g in range(3):
            _, slots1 = _steps(GROUP_MASKS[g])[0]
            recv_add(g, 0, 2, slots1[2])
            recv_add(g, 0, 3, slots1[3])

        for g in range(3):
            _, slots2 = _steps(GROUP_MASKS[g])[1]
            recv_add(g, 1, 0, slots2[0])
            mk(g, 2, 0, 0).start()

        for g in range(3):
            _, slots2 = _steps(GROUP_MASKS[g])[1]
            recv_add(g, 1, 1, slots2[1])

        for g, (c0, w) in enumerate(GROUP_COLS):
            recv_add(g, 2, 0, 0)
            out_ref[:, c0:c0 + w] = _gelu(
                accs[g][pl.ds(my * m_per, m_per), :]
            )

        for g in range(3):
            for step, (md, slots) in enumerate(_steps(GROUP_MASKS[g])):
                for j, s in enumerate(slots):
                    mk(g, step, j, s).wait_send()

    scratch = []
    for c0, w in GROUP_COLS:
        scratch.append(pltpu.VMEM((m, w), jnp.float32))
    for c0, w in GROUP_COLS:
        scratch.append(pltpu.VMEM((N_SLOT, m_per, w), jnp.float32))
    for _ in range(3):
        scratch.append(pltpu.SemaphoreType.DMA((N_SLOT,)))
        scratch.append(pltpu.SemaphoreType.DMA((N_SLOT,)))

    def reorder(x_ref, w_ref, out_ref, a0, a1, a2, b0, b1, b2,
                s0, r0, s1, r1, s2, r2):
        return body(x_ref, w_ref, out_ref, a0, a1, a2, b0, b1, b2,
                    s0, r0, s1, r1, s2, r2)

    return pl.pallas_call(
        reorder,
        out_shape=jax.ShapeDtypeStruct((m_per, n), jnp.float32),
        in_specs=[
            pl.BlockSpec(memory_space=pltpu.VMEM),
            pl.BlockSpec(memory_space=pltpu.VMEM),
        ],
        out_specs=pl.BlockSpec(memory_space=pltpu.VMEM),
        scratch_shapes=scratch,
        compiler_params=pltpu.CompilerParams(collective_id=0),
    )(x, w_mat)
